# Initial kernel scaffold; baseline (speedup 1.0000x reference)
import jax
import jax.numpy as jnp
from jax.experimental import pallas as pl


def kernel(x, edge_index, edge_attr, u, batch, W1, b1, W2, b2):
    raise NotImplementedError("write your pallas kernel here")


# speedup vs baseline: 3.1377x; 3.1377x over previous
import functools

import jax
import jax.numpy as jnp
from jax import lax
from jax.experimental import pallas as pl
from jax.experimental.pallas import tpu as pltpu
from jax.experimental.pallas import tpu_sc as plsc

N_NODES = 10000
D = 128
G = 64
NC = 2
NS = 16
NW = NC * NS
ROWS_PER = 312
ROWS_MAIN = NW * ROWS_PER
CHUNK = 104
N_CHUNKS = ROWS_PER // CHUNK
TAIL = N_NODES - ROWS_MAIN


def _sc_segment_sum(x, batch2d, batch_tail, zeros):
    mesh = plsc.VectorSubcoreMesh(core_axis_name="c", subcore_axis_name="s")

    @functools.partial(
        pl.kernel,
        mesh=mesh,
        out_type=jax.ShapeDtypeStruct((NC, G, D), jnp.float32),
        scratch_types=[
            pltpu.VMEM((ROWS_PER, D), jnp.float32),
            pltpu.VMEM((N_CHUNKS, CHUNK), jnp.int32),
            pltpu.VMEM((TAIL, D), jnp.float32),
            pltpu.VMEM((1, TAIL), jnp.int32),
            pltpu.VMEM_SHARED((G, D), jnp.float32),
        ],
    )
    def seg_sum(x_hbm, b2d_hbm, btail_hbm, zeros_hbm, out_hbm,
                xbuf, idxbuf, tbuf, tidx, acc):
        c = lax.axis_index("c")
        s = lax.axis_index("s")
        wid = s * NC + c

        @pl.when(s == 0)
        def _():
            pltpu.sync_copy(zeros_hbm, acc)

        plsc.subcore_barrier()

        base = wid * ROWS_PER
        pltpu.sync_copy(x_hbm.at[pl.ds(base, ROWS_PER)], xbuf)
        pltpu.sync_copy(b2d_hbm.at[wid], idxbuf)
        for j in range(N_CHUNKS):
            pltpu.sync_copy(xbuf.at[pl.ds(j * CHUNK, CHUNK)],
                            acc.at[idxbuf.at[j]], add=True)

        @pl.when(wid == NW - 1)
        def _():
            pltpu.sync_copy(x_hbm.at[pl.ds(ROWS_MAIN, TAIL)], tbuf)
            pltpu.sync_copy(btail_hbm, tidx)
            pltpu.sync_copy(tbuf, acc.at[tidx.at[0]], add=True)

        plsc.subcore_barrier()

        @pl.when(s == 0)
        def _():
            pltpu.sync_copy(acc, out_hbm.at[c])

    return seg_sum(x, batch2d, batch_tail, zeros)


def _tc_mlp(partials, u, W1, b1, W2, b2):
    def body(p_ref, u_ref, w1_ref, b1_ref, w2_ref, b2_ref, o_ref):
        pooled = p_ref[0] + p_ref[1]
        out = jnp.concatenate([u_ref[...], pooled], axis=1)
        h = jnp.dot(out, w1_ref[...], preferred_element_type=jnp.float32)
        h = jnp.maximum(h + b1_ref[...], 0.0)
        o_ref[...] = (jnp.dot(h, w2_ref[...], preferred_element_type=jnp.float32)
                      + b2_ref[...])

    return pl.pallas_call(
        body,
        out_shape=jax.ShapeDtypeStruct((G, 128), jnp.float32),
    )(partials, u, W1, b1.reshape(1, -1), W2, b2.reshape(1, -1))


def kernel(x, edge_index, edge_attr, u, batch, W1, b1, W2, b2):
    batch2d = batch[:ROWS_MAIN].reshape(NW, N_CHUNKS, CHUNK)
    batch_tail = batch[ROWS_MAIN:].reshape(1, TAIL)
    zeros = jnp.zeros((G, D), jnp.float32)
    partials = _sc_segment_sum(x, batch2d, batch_tail, zeros)
    return _tc_mlp(partials, u, W1, b1, W2, b2)


# speedup vs baseline: 3.3709x; 1.0743x over previous
import functools

import jax
import jax.numpy as jnp
from jax import lax
from jax.experimental import pallas as pl
from jax.experimental.pallas import tpu as pltpu
from jax.experimental.pallas import tpu_sc as plsc

N_NODES = 10000
D = 128
G = 64
NC = 2
NS = 16
NW = NC * NS
ROWS_PER = 312
ROWS_MAIN = NW * ROWS_PER
CHUNK = 104
N_CHUNKS = ROWS_PER // CHUNK
TAIL = N_NODES - ROWS_MAIN


def _sc_segment_sum(x, batch2d, batch_tail, zeros):
    mesh = plsc.VectorSubcoreMesh(core_axis_name="c", subcore_axis_name="s")

    @functools.partial(
        pl.kernel,
        mesh=mesh,
        out_type=jax.ShapeDtypeStruct((NC, G, D), jnp.float32),
        scratch_types=[
            pltpu.VMEM((ROWS_PER, D), jnp.float32),
            pltpu.VMEM((N_CHUNKS, CHUNK), jnp.int32),
            pltpu.VMEM((TAIL, D), jnp.float32),
            pltpu.VMEM((1, TAIL), jnp.int32),
            pltpu.VMEM_SHARED((G, D), jnp.float32),
            pltpu.SemaphoreType.DMA,
            pltpu.SemaphoreType.DMA,
            pltpu.SemaphoreType.DMA,
            pltpu.SemaphoreType.DMA,
            pltpu.SemaphoreType.DMA,
        ],
    )
    def seg_sum(x_hbm, b2d_hbm, btail_hbm, zeros_hbm, out_hbm,
                xbuf, idxbuf, tbuf, tidx, acc,
                sem_g0, sem_g1, sem_g2, sem_idx, sem_sc):
        c = lax.axis_index("c")
        s = lax.axis_index("s")
        wid = s * NC + c
        base = wid * ROWS_PER
        gsems = [sem_g0, sem_g1, sem_g2]

        idx_cp = pltpu.async_copy(b2d_hbm.at[wid], idxbuf, sem_idx)
        gathers = [
            pltpu.async_copy(x_hbm.at[pl.ds(base + j * CHUNK, CHUNK)],
                             xbuf.at[pl.ds(j * CHUNK, CHUNK)], gsems[j])
            for j in range(N_CHUNKS)
        ]

        @pl.when(s < 8)
        def _():
            pltpu.sync_copy(zeros_hbm.at[pl.ds(s * 8, 8)],
                            acc.at[pl.ds(s * 8, 8)])

        plsc.subcore_barrier()

        idx_cp.wait()
        scatters = []
        for j in range(N_CHUNKS):
            gathers[j].wait()
            scatters.append(
                pltpu.async_copy(xbuf.at[pl.ds(j * CHUNK, CHUNK)],
                                 acc.at[idxbuf.at[j]], sem_sc, add=True))

        @pl.when(wid == NW - 1)
        def _():
            pltpu.sync_copy(x_hbm.at[pl.ds(ROWS_MAIN, TAIL)], tbuf)
            pltpu.sync_copy(btail_hbm, tidx)
            pltpu.sync_copy(tbuf, acc.at[tidx.at[0]], add=True)

        for cp in scatters:
            cp.wait()

        plsc.subcore_barrier()

        @pl.when(s == 0)
        def _():
            pltpu.sync_copy(acc, out_hbm.at[c])

    return seg_sum(x, batch2d, batch_tail, zeros)


def _tc_mlp(partials, u, W1, b1, W2, b2):
    def body(p_ref, u_ref, w1_ref, b1_ref, w2_ref, b2_ref, o_ref):
        pooled = p_ref[0] + p_ref[1]
        out = jnp.concatenate([u_ref[...], pooled], axis=1)
        h = jnp.dot(out, w1_ref[...], preferred_element_type=jnp.float32)
        h = jnp.maximum(h + b1_ref[...], 0.0)
        o_ref[...] = (jnp.dot(h, w2_ref[...], preferred_element_type=jnp.float32)
                      + b2_ref[...])

    return pl.pallas_call(
        body,
        out_shape=jax.ShapeDtypeStruct((G, 128), jnp.float32),
    )(partials, u, W1, b1.reshape(1, -1), W2, b2.reshape(1, -1))


def kernel(x, edge_index, edge_attr, u, batch, W1, b1, W2, b2):
    batch2d = batch[:ROWS_MAIN].reshape(NW, N_CHUNKS, CHUNK)
    batch_tail = batch[ROWS_MAIN:].reshape(1, TAIL)
    zeros = jnp.zeros((G, D), jnp.float32)
    partials = _sc_segment_sum(x, batch2d, batch_tail, zeros)
    return _tc_mlp(partials, u, W1, b1, W2, b2)


# speedup vs baseline: 3.4557x; 1.0251x over previous
import functools

import jax
import jax.numpy as jnp
from jax import lax
from jax.experimental import pallas as pl
from jax.experimental.pallas import tpu as pltpu
from jax.experimental.pallas import tpu_sc as plsc

N_NODES = 10000
D = 128
G = 64
NC = 2
NS = 16
NW = NC * NS
ROWS_PER = 312
ROWS_MAIN = NW * ROWS_PER
CHUNK = 104
N_CHUNKS = ROWS_PER // CHUNK
TAIL = N_NODES - ROWS_MAIN


def _sc_segment_sum(x, batch):
    mesh = plsc.VectorSubcoreMesh(core_axis_name="c", subcore_axis_name="s")

    @functools.partial(
        pl.kernel,
        mesh=mesh,
        out_type=jax.ShapeDtypeStruct((NC, G, D), jnp.float32),
        scratch_types=[
            pltpu.VMEM((ROWS_PER, D), jnp.float32),
            pltpu.VMEM((N_CHUNKS, CHUNK), jnp.int32),
            pltpu.VMEM((TAIL, D), jnp.float32),
            pltpu.VMEM((1, TAIL), jnp.int32),
            pltpu.VMEM((8, D), jnp.float32),
            pltpu.VMEM_SHARED((G, D), jnp.float32),
            pltpu.SemaphoreType.DMA,
            pltpu.SemaphoreType.DMA,
            pltpu.SemaphoreType.DMA,
            pltpu.SemaphoreType.DMA,
            pltpu.SemaphoreType.DMA,
        ],
    )
    def seg_sum(x_hbm, b_hbm, out_hbm,
                xbuf, idxbuf, tbuf, tidx, zbuf, acc,
                sem_g0, sem_g1, sem_g2, sem_idx, sem_sc):
        c = lax.axis_index("c")
        s = lax.axis_index("s")
        wid = s * NC + c
        base = wid * ROWS_PER
        gsems = [sem_g0, sem_g1, sem_g2]

        idx_cps = [
            pltpu.async_copy(b_hbm.at[pl.ds(base + j * CHUNK, CHUNK)],
                             idxbuf.at[j], sem_idx)
            for j in range(N_CHUNKS)
        ]
        gathers = [
            pltpu.async_copy(x_hbm.at[pl.ds(base + j * CHUNK, CHUNK)],
                             xbuf.at[pl.ds(j * CHUNK, CHUNK)], gsems[j])
            for j in range(N_CHUNKS)
        ]

        @pl.when(s < 8)
        def _():
            zero16 = jnp.zeros((16,), jnp.float32)
            for r in range(8):
                for k in range(D // 16):
                    zbuf[r, pl.ds(k * 16, 16)] = zero16
            pltpu.sync_copy(zbuf, acc.at[pl.ds(s * 8, 8)])

        plsc.subcore_barrier()

        for cp in idx_cps:
            cp.wait()
        scatters = []
        for j in range(N_CHUNKS):
            gathers[j].wait()
            scatters.append(
                pltpu.async_copy(xbuf.at[pl.ds(j * CHUNK, CHUNK)],
                                 acc.at[idxbuf.at[j]], sem_sc, add=True))

        @pl.when(wid == NW - 1)
        def _():
            pltpu.sync_copy(x_hbm.at[pl.ds(ROWS_MAIN, TAIL)], tbuf)
            pltpu.sync_copy(b_hbm.at[pl.ds(ROWS_MAIN, TAIL)], tidx.at[0])
            pltpu.sync_copy(tbuf, acc.at[tidx.at[0]], add=True)

        for cp in scatters:
            cp.wait()

        plsc.subcore_barrier()

        @pl.when(s == 0)
        def _():
            pltpu.sync_copy(acc, out_hbm.at[c])

    return seg_sum(x, batch)


def _tc_mlp(partials, u, W1, b1, W2, b2):
    def body(p_ref, u_ref, w1_ref, b1_ref, w2_ref, b2_ref, o_ref):
        pooled = p_ref[0] + p_ref[1]
        out = jnp.concatenate([u_ref[...], pooled], axis=1)
        h = jnp.dot(out, w1_ref[...], preferred_element_type=jnp.float32)
        h = jnp.maximum(h + b1_ref[...], 0.0)
        o_ref[...] = (jnp.dot(h, w2_ref[...], preferred_element_type=jnp.float32)
                      + b2_ref[...])

    return pl.pallas_call(
        body,
        out_shape=jax.ShapeDtypeStruct((G, 128), jnp.float32),
    )(partials, u, W1, b1.reshape(1, -1), W2, b2.reshape(1, -1))


def kernel(x, edge_index, edge_attr, u, batch, W1, b1, W2, b2):
    partials = _sc_segment_sum(x, batch)
    return _tc_mlp(partials, u, W1, b1, W2, b2)
